# Initial kernel scaffold; baseline (speedup 1.0000x reference)
import jax
import jax.numpy as jnp
from jax.experimental import pallas as pl


def kernel(x, I):
    raise NotImplementedError("write your pallas kernel here")


# speedup vs baseline: 8.5004x; 8.5004x over previous
import jax
import jax.numpy as jnp
from jax import lax
from jax.experimental import pallas as pl
from jax.experimental.pallas import tpu as pltpu
from jax.experimental.pallas import tpu_sc as plsc

_C, _G, _S, _L = 16, 4096, 8, 2
_B = 32
_STEPS = 3
_GAMMA = 0.01
_IG = 100.0
_LN2 = 0.6931471805599453

_NC, _NS = 2, 16
_NW = _NC * _NS
_SLOTS = _C * _G
_SPW = _SLOTS // _NW
_CS = 8
_RPC = _CS * _S * _L
_CPW = _SPW // _CS
_OSL = 256
_CPO = _OSL // _CS
_OBW = _SPW // _OSL
_NROWS = _SLOTS * _S * _L // _RPC


def _sc_log(acc):
    bits = lax.bitcast_convert_type(acc, jnp.int32)
    e = ((bits >> 23) & 0xFF) - 127
    man = lax.bitcast_convert_type((bits & 0x007FFFFF) | 0x3F800000,
                                   jnp.float32)
    t = man - 1.0
    s = t / (t + 2.0)
    s2 = s * s
    p = 1.0 + s2 * (jnp.float32(1.0 / 3.0)
                    + s2 * (jnp.float32(0.2) + s2 * jnp.float32(1.0 / 7.0)))
    return e.astype(jnp.float32) * jnp.float32(_LN2) + 2.0 * s * p


def _tree_max(vals):
    while len(vals) > 1:
        vals = [jnp.maximum(vals[i], vals[i + 1])
                for i in range(0, len(vals) - 1, 2)] + (
                    [vals[-1]] if len(vals) % 2 else [])
    return vals[0]


def _stage_a_body(idx_hbm, xt_hbm, p_hbm, mx_hbm,
                  idx_v, rows_v, out_v, mxv, sem):
    cid = lax.axis_index("c")
    sid = lax.axis_index("s")
    w = sid * _NC + cid
    cc = w // 2
    gb = (w % 2) * (_G // 2)
    neg = jnp.full((16,), -3.0e38, jnp.float32)

    def chunk_body(ob, ch, carry):
        rm0, rm1 = carry
        row = w * _CPW + ob * _CPO + ch
        pltpu.sync_copy(idx_hbm.at[pl.ds(row * _RPC, _RPC)], idx_v)
        pltpu.async_copy(xt_hbm.at[idx_v], rows_v, sem).wait()
        col = ch * _CS
        for k in range(_CS):
            base = k * 16
            for h in range(2):
                lo = h * 16
                rs = [rows_v[base + j, pl.ds(lo, 16)] for j in range(16)]
                bs = [rs[2 * s] * rs[2 * s + 1] for s in range(_S)]
                m = _tree_max(bs)
                acc = jnp.exp((bs[0] - m) * _IG)
                for b in bs[1:]:
                    acc = acc + jnp.exp((b - m) * _IG)
                lse = m + _GAMMA * _sc_log(acc)
                out_v[pl.ds((col + k) * _B + lo, 16)] = lse
                if h == 0:
                    rm0 = jnp.maximum(rm0, lse)
                else:
                    rm1 = jnp.maximum(rm1, lse)
        return rm0, rm1

    def ob_body(ob, carry):
        carry = lax.fori_loop(0, _CPO,
                              lambda ch, c_: chunk_body(ob, ch, c_), carry)
        pltpu.sync_copy(
            out_v,
            p_hbm.at[pl.ds(((cc * _G + gb) + ob * _OSL) * _B, _OSL * _B)])
        return carry

    rm = lax.fori_loop(0, _OBW, ob_body, (neg, neg))
    mxv[...] = jnp.maximum(rm[0], rm[1])
    pltpu.sync_copy(mxv, mx_hbm.at[pl.ds(w * 16, 16)])


_stage_a = pl.kernel(
    _stage_a_body,
    out_type=(jax.ShapeDtypeStruct((_C * _G * _B,), jnp.float32),
              jax.ShapeDtypeStruct((_NW * 16,), jnp.float32)),
    mesh=plsc.VectorSubcoreMesh(core_axis_name="c", subcore_axis_name="s"),
    compiler_params=pltpu.CompilerParams(use_tc_tiling_on_sc=False),
    scratch_types=(
        pltpu.VMEM((_RPC,), jnp.int32),
        pltpu.VMEM((_RPC, _B), jnp.float32),
        pltpu.VMEM((_OSL * _B,), jnp.float32),
        pltpu.VMEM((16,), jnp.float32),
        pltpu.SemaphoreType.DMA,
    ),
)


def _stage_b_body(p_ref, mx_ref, rt_ref, rnt_ref):
    m1 = jnp.max(mx_ref[...])
    s1 = jnp.where(m1 > 1.0, 1.0 / m1, 1.0)
    cv = p_ref[...] * s1
    mxc = jnp.max(cv, axis=0)
    acc = jnp.sum(jnp.exp((cv - mxc[None, :, :]) * _IG), axis=0)
    lse_c = mxc + _GAMMA * jnp.log(acc)
    m2 = jnp.max(lse_c)
    rr = lse_c * jnp.where(m2 > 1.0, 1.0 / m2, 1.0)
    rc = rt_ref[...]
    mx2 = jnp.maximum(rc, rr)
    z = mx2 + _GAMMA * jnp.log(jnp.exp((rc - mx2) * _IG)
                               + jnp.exp((rr - mx2) * _IG))
    m3 = jnp.max(z)
    rnt_ref[...] = z * jnp.where(m3 > 1.0, 1.0 / m3, 1.0)


_GB = _G * _B
_ROWS128 = _GB // 128

_stage_b = pl.pallas_call(
    _stage_b_body,
    out_shape=jax.ShapeDtypeStruct((_ROWS128, 128), jnp.float32),
)


def _tr_body(rt_ref, r_ref):
    r_ref[...] = rt_ref[...].T


_tr = pl.pallas_call(
    _tr_body,
    out_shape=jax.ShapeDtypeStruct((_B, _G), jnp.float32),
)


def kernel(x, I):
    idx = I.reshape(_NROWS * _RPC).astype(jnp.int32)
    rt = x.T
    for _ in range(_STEPS):
        p, mx = _stage_a(idx, rt)
        rtf = _stage_b(p.reshape(_C, _ROWS128, 128), mx.reshape(_NW, 16),
                       rt.reshape(_ROWS128, 128))
        rt = rtf.reshape(_G, _B)
    return _tr(rt)


# speedup vs baseline: 9.1919x; 1.0814x over previous
import jax
import jax.numpy as jnp
from jax import lax
from jax.experimental import pallas as pl
from jax.experimental.pallas import tpu as pltpu
from jax.experimental.pallas import tpu_sc as plsc

_C, _G, _S, _L = 16, 4096, 8, 2
_B = 32
_STEPS = 3
_GAMMA = 0.01
_IG = 100.0
_LN2 = 0.6931471805599453

_NC, _NS = 2, 16
_NW = _NC * _NS
_SLOTS = _C * _G
_SPW = _SLOTS // _NW
_CS = 8
_RPC = _CS * _S * _L
_CPW = _SPW // _CS
_OSL = 256
_CPO = _OSL // _CS
_OBW = _SPW // _OSL
_NROWS = _SLOTS * _S * _L // _RPC


def _sc_log(acc):
    bits = lax.bitcast_convert_type(acc, jnp.int32)
    e = ((bits >> 23) & 0xFF) - 127
    man = lax.bitcast_convert_type((bits & 0x007FFFFF) | 0x3F800000,
                                   jnp.float32)
    t = man - 1.0
    s = t / (t + 2.0)
    s2 = s * s
    p = 1.0 + s2 * (jnp.float32(1.0 / 3.0)
                    + s2 * (jnp.float32(0.2) + s2 * jnp.float32(1.0 / 7.0)))
    return e.astype(jnp.float32) * jnp.float32(_LN2) + 2.0 * s * p


def _tree_max(vals):
    while len(vals) > 1:
        vals = [jnp.maximum(vals[i], vals[i + 1])
                for i in range(0, len(vals) - 1, 2)] + (
                    [vals[-1]] if len(vals) % 2 else [])
    return vals[0]


def _compute_chunk(rows_v, out_v, col):
    for k in range(_CS):
        base = k * 16
        for h in range(2):
            lo = h * 16
            rs = [rows_v[base + j, pl.ds(lo, 16)] for j in range(16)]
            bs = [rs[2 * s] * rs[2 * s + 1] for s in range(_S)]
            m = _tree_max(bs)
            acc = jnp.exp((bs[0] - m) * _IG)
            for b in bs[1:]:
                acc = acc + jnp.exp((b - m) * _IG)
            lse = m + _GAMMA * _sc_log(acc)
            out_v[pl.ds((col + k) * _B + lo, 16)] = lse


def _stage_a_body(idx_hbm, xt_hbm, p_hbm,
                  idx_v, rows_a, rows_b, out_v, sem_a, sem_b):
    cid = lax.axis_index("c")
    sid = lax.axis_index("s")
    w = sid * _NC + cid
    cc = w // 2
    gb = (w % 2) * (_G // 2)

    pltpu.sync_copy(idx_hbm.at[pl.ds(w * _CPW, _CPW), :], idx_v)

    def issue(ch, rows, sem):
        pltpu.async_copy(xt_hbm.at[idx_v.at[ch]], rows, sem)

    def wait(rows, sem):
        pltpu.make_async_copy(xt_hbm.at[idx_v.at[0]], rows, sem).wait()

    def ob_body(ob, carry):
        c0 = ob * _CPO
        issue(c0, rows_a, sem_a)

        def pair_body(p, c_):
            j0 = c0 + p * 2
            issue(j0 + 1, rows_b, sem_b)
            wait(rows_a, sem_a)
            _compute_chunk(rows_a, out_v, (p * 2) * _CS)

            @pl.when(p < _CPO // 2 - 1)
            def _():
                issue(j0 + 2, rows_a, sem_a)

            wait(rows_b, sem_b)
            _compute_chunk(rows_b, out_v, (p * 2 + 1) * _CS)
            return c_

        lax.fori_loop(0, _CPO // 2, pair_body, 0)
        pltpu.sync_copy(
            out_v,
            p_hbm.at[pl.ds(((cc * _G + gb) + ob * _OSL) * _B, _OSL * _B)])
        return carry

    lax.fori_loop(0, _OBW, ob_body, 0)


_stage_a = pl.kernel(
    _stage_a_body,
    out_type=jax.ShapeDtypeStruct((_C * _G * _B,), jnp.float32),
    mesh=plsc.VectorSubcoreMesh(core_axis_name="c", subcore_axis_name="s"),
    compiler_params=pltpu.CompilerParams(use_tc_tiling_on_sc=False),
    scratch_types=(
        pltpu.VMEM((_CPW, _RPC), jnp.int32),
        pltpu.VMEM((_RPC, _B), jnp.float32),
        pltpu.VMEM((_RPC, _B), jnp.float32),
        pltpu.VMEM((_OSL * _B,), jnp.float32),
        pltpu.SemaphoreType.DMA,
        pltpu.SemaphoreType.DMA,
    ),
)


def _stage_b_body(p_ref, rt_ref, rnt_ref):
    m1 = jnp.max(p_ref[...])
    s1 = jnp.where(m1 > 1.0, 1.0 / m1, 1.0)
    cv = p_ref[...] * s1
    mxc = jnp.max(cv, axis=0)
    acc = jnp.sum(jnp.exp((cv - mxc[None, :, :]) * _IG), axis=0)
    lse_c = mxc + _GAMMA * jnp.log(acc)
    m2 = jnp.max(lse_c)
    rr = lse_c * jnp.where(m2 > 1.0, 1.0 / m2, 1.0)
    rc = rt_ref[...]
    mx2 = jnp.maximum(rc, rr)
    z = mx2 + _GAMMA * jnp.log(jnp.exp((rc - mx2) * _IG)
                               + jnp.exp((rr - mx2) * _IG))
    m3 = jnp.max(z)
    rnt_ref[...] = z * jnp.where(m3 > 1.0, 1.0 / m3, 1.0)


_GB = _G * _B
_ROWS128 = _GB // 128

_stage_b = pl.pallas_call(
    _stage_b_body,
    out_shape=jax.ShapeDtypeStruct((_ROWS128, 128), jnp.float32),
)


def _tr_body(rt_ref, r_ref):
    r_ref[...] = rt_ref[...].T


_tr = pl.pallas_call(
    _tr_body,
    out_shape=jax.ShapeDtypeStruct((_B, _G), jnp.float32),
)


def kernel(x, I):
    idx = I.reshape(_NROWS, _RPC).astype(jnp.int32)
    rt = x.T
    for _ in range(_STEPS):
        p = _stage_a(idx, rt)
        rtf = _stage_b(p.reshape(_C, _ROWS128, 128),
                       rt.reshape(_ROWS128, 128))
        rt = rtf.reshape(_G, _B)
    return _tr(rt)


# speedup vs baseline: 11.4185x; 1.2422x over previous
import jax
import jax.numpy as jnp
from jax import lax
from jax.experimental import pallas as pl
from jax.experimental.pallas import tpu as pltpu
from jax.experimental.pallas import tpu_sc as plsc

_C, _G, _S, _L = 16, 4096, 8, 2
_B = 32
_STEPS = 3
_GAMMA = 0.01
_IG = 100.0
_LN2 = 0.6931471805599453

_NC, _NS = 2, 16
_NW = _NC * _NS
_SLOTS = _C * _G
_SPW = _SLOTS // _NW
_CS = 8
_RPC = _CS * _S * _L
_CPW = _SPW // _CS
_OSL = 256
_CPO = _OSL // _CS
_OBW = _SPW // _OSL
_NROWS = _SLOTS * _S * _L // _RPC


def _p1(f, xs, *cs):
    return [f(x, *cs) for x in xs]


def _p2(f, xs, ys):
    return [f(x, y) for x, y in zip(xs, ys)]


def _ptree(f, pairs_list):
    while len(pairs_list) > 1:
        nxt = [_p2(f, pairs_list[i], pairs_list[i + 1])
               for i in range(0, len(pairs_list) - 1, 2)]
        if len(pairs_list) % 2:
            nxt.append(pairs_list[-1])
        pairs_list = nxt
    return pairs_list[0]


def _sc_log_pair(accs):
    bits = _p1(lambda a: lax.bitcast_convert_type(a, jnp.int32), accs)
    e = _p1(lambda b: ((b >> 23) & 0xFF) - 127, bits)
    man = _p1(lambda b: lax.bitcast_convert_type(
        (b & 0x007FFFFF) | 0x3F800000, jnp.float32), bits)
    t = _p1(lambda mn: mn - 1.0, man)
    s = _p2(lambda tt, d: tt / d, t, _p1(lambda tt: tt + 2.0, t))
    s2 = _p2(lambda a, b: a * b, s, s)
    p = _p1(lambda q: jnp.float32(0.2) + q * jnp.float32(1.0 / 7.0), s2)
    p = _p2(lambda q, pp: jnp.float32(1.0 / 3.0) + q * pp, s2, p)
    p = _p2(lambda q, pp: 1.0 + q * pp, s2, p)
    ef = _p1(lambda ee: ee.astype(jnp.float32) * jnp.float32(_LN2), e)
    sp = _p2(lambda a, b: 2.0 * a * b, s, p)
    return _p2(lambda a, b: a + b, ef, sp)


def _compute_chunk(rows_v, out_v, col):
    for k in range(_CS):
        base = k * 16
        rs = [[rows_v[base + j, pl.ds(lo, 16)] for lo in (0, 16)]
              for j in range(16)]
        bs = [_p2(lambda a, b: a * b, rs[2 * s], rs[2 * s + 1])
              for s in range(_S)]
        m = _ptree(jnp.maximum, bs)
        es = [_p2(lambda b, mm: jnp.exp((b - mm) * _IG), b, m) for b in bs]
        acc = _ptree(lambda a, b: a + b, es)
        ln = _sc_log_pair(acc)
        lse = _p2(lambda mm, l_: mm + _GAMMA * l_, m, ln)
        out_v[pl.ds((col + k) * _B, 16)] = lse[0]
        out_v[pl.ds((col + k) * _B + 16, 16)] = lse[1]


def _stage_a_body(idx_hbm, xt_hbm, p_hbm,
                  idx_v, rows_a, rows_b, out_v, sem_a, sem_b):
    cid = lax.axis_index("c")
    sid = lax.axis_index("s")
    w = sid * _NC + cid
    cc = w // 2
    gb = (w % 2) * (_G // 2)

    pltpu.sync_copy(idx_hbm.at[pl.ds(w * _CPW, _CPW), :], idx_v)

    def issue(ch, rows, sem):
        pltpu.async_copy(xt_hbm.at[idx_v.at[ch]], rows, sem)

    def wait(rows, sem):
        pltpu.make_async_copy(xt_hbm.at[idx_v.at[0]], rows, sem).wait()

    def ob_body(ob, carry):
        c0 = ob * _CPO
        issue(c0, rows_a, sem_a)

        def pair_body(p, c_):
            j0 = c0 + p * 2
            issue(j0 + 1, rows_b, sem_b)
            wait(rows_a, sem_a)
            _compute_chunk(rows_a, out_v, (p * 2) * _CS)

            @pl.when(p < _CPO // 2 - 1)
            def _():
                issue(j0 + 2, rows_a, sem_a)

            wait(rows_b, sem_b)
            _compute_chunk(rows_b, out_v, (p * 2 + 1) * _CS)
            return c_

        lax.fori_loop(0, _CPO // 2, pair_body, 0)
        pltpu.sync_copy(
            out_v,
            p_hbm.at[pl.ds(((cc * _G + gb) + ob * _OSL) * _B, _OSL * _B)])
        return carry

    lax.fori_loop(0, _OBW, ob_body, 0)


_stage_a = pl.kernel(
    _stage_a_body,
    out_type=jax.ShapeDtypeStruct((_C * _G * _B,), jnp.float32),
    mesh=plsc.VectorSubcoreMesh(core_axis_name="c", subcore_axis_name="s"),
    compiler_params=pltpu.CompilerParams(use_tc_tiling_on_sc=False),
    scratch_types=(
        pltpu.VMEM((_CPW, _RPC), jnp.int32),
        pltpu.VMEM((_RPC, _B), jnp.float32),
        pltpu.VMEM((_RPC, _B), jnp.float32),
        pltpu.VMEM((_OSL * _B,), jnp.float32),
        pltpu.SemaphoreType.DMA,
        pltpu.SemaphoreType.DMA,
    ),
)


def _stage_b_body(p_ref, rt_ref, rnt_ref):
    m1 = jnp.max(p_ref[...])
    s1 = jnp.where(m1 > 1.0, 1.0 / m1, 1.0)
    cv = p_ref[...] * s1
    mxc = jnp.max(cv, axis=0)
    acc = jnp.sum(jnp.exp((cv - mxc[None, :, :]) * _IG), axis=0)
    lse_c = mxc + _GAMMA * jnp.log(acc)
    m2 = jnp.max(lse_c)
    rr = lse_c * jnp.where(m2 > 1.0, 1.0 / m2, 1.0)
    rc = rt_ref[...]
    mx2 = jnp.maximum(rc, rr)
    z = mx2 + _GAMMA * jnp.log(jnp.exp((rc - mx2) * _IG)
                               + jnp.exp((rr - mx2) * _IG))
    m3 = jnp.max(z)
    rnt_ref[...] = z * jnp.where(m3 > 1.0, 1.0 / m3, 1.0)


_GB = _G * _B
_ROWS128 = _GB // 128

_stage_b = pl.pallas_call(
    _stage_b_body,
    out_shape=jax.ShapeDtypeStruct((_ROWS128, 128), jnp.float32),
)


def _tr_body(rt_ref, r_ref):
    r_ref[...] = rt_ref[...].T


_tr = pl.pallas_call(
    _tr_body,
    out_shape=jax.ShapeDtypeStruct((_B, _G), jnp.float32),
)


def kernel(x, I):
    idx = I.reshape(_NROWS, _RPC).astype(jnp.int32)
    rt = x.T
    for _ in range(_STEPS):
        p = _stage_a(idx, rt)
        rtf = _stage_b(p.reshape(_C, _ROWS128, 128),
                       rt.reshape(_ROWS128, 128))
        rt = rtf.reshape(_G, _B)
    return _tr(rt)
